# Initial kernel scaffold; baseline (speedup 1.0000x reference)
import jax
import jax.numpy as jnp
from jax.experimental import pallas as pl


def kernel(x, edge_index, W1, b1, W2, b2):
    raise NotImplementedError("write your pallas kernel here")


# speedup vs baseline: 9.5520x; 9.5520x over previous
import functools
import math

import jax
import jax.numpy as jnp
from jax import lax
from jax.experimental import pallas as pl
from jax.experimental.pallas import tpu as pltpu
from jax.experimental.pallas import tpu_sc as plsc

_N = 10000
_E_REAL = 330000
_H = 8
_C1 = 16
_C2 = 40
_C2P = 48
_D1 = _H * _C1
_D2 = _H * _C2P

_NC = 2
_NS = 16
_L = 16
_NW = _NC * _NS

_B = 128
_CH = 81
_EW = _B * _CH
_EPAD = _EW * _NW
_NPAD = 10240
_RPT = _NPAD // _NS

_SCALE1 = 0.25
_SCALE2 = 1.0 / math.sqrt(40.0)

_f32 = jnp.float32
_i32 = jnp.int32


def _mesh():
    return plsc.VectorSubcoreMesh(core_axis_name="c", subcore_axis_name="s")


def _make_pass1(D, nsub, scale):
    hs = nsub * _L

    @functools.partial(
        pl.kernel,
        out_type=(
            jax.ShapeDtypeStruct((_EPAD, 16), _f32),
            jax.ShapeDtypeStruct((_NC, _NPAD, 16), _f32),
        ),
        mesh=_mesh(),
        compiler_params=pltpu.CompilerParams(needs_layout_passes=False, use_tc_tiling_on_sc=False),
        scratch_types=[
            pltpu.VMEM((_B,), _i32),
            pltpu.VMEM((_B,), _i32),
            pltpu.VMEM((_B, D), _f32),
            pltpu.VMEM((_B, D), _f32),
            pltpu.VMEM((_B, 16), _f32),
            pltpu.VMEM_SHARED((_NPAD, 16), _f32),
            pltpu.SemaphoreType.DMA,
        ],
    )
    def pass1(xw, src, dst, zr, p_out, dpart, sidx, didx, srows, drows,
              ebuf, dsp, sem):
        cid = lax.axis_index("c")
        sid = lax.axis_index("s")
        wid = sid * _NC + cid
        pltpu.sync_copy(zr, dsp.at[pl.ds(sid * _RPT, _RPT)])
        plsc.subcore_barrier()
        lanes = lax.iota(_i32, _L)

        def zrow(b, c2):
            ebuf[b, :] = jnp.zeros((_L,), _f32)
            return c2

        lax.fori_loop(0, _B, zrow, 0)
        hvecs = [jnp.full((_L,), h, _i32) for h in range(_H)]

        def chunk(j, carry):
            ebase = wid * _EW + j * _B
            pltpu.sync_copy(src.at[pl.ds(ebase, _B)], sidx)
            pltpu.sync_copy(dst.at[pl.ds(ebase, _B)], didx)
            pltpu.async_copy(xw.at[sidx], srows, sem).wait()
            pltpu.async_copy(xw.at[didx], drows, sem).wait()

            def group(g, c2):
                bvec = g * _L + lanes
                for h in range(_H):
                    acc = jnp.zeros((_L,), _f32)
                    cvec = jnp.full((_L,), h * hs, _i32)
                    for _c in range(hs):
                        sv = plsc.load_gather(srows, [bvec, cvec])
                        dv = plsc.load_gather(drows, [bvec, cvec])
                        acc = acc + sv * dv
                        cvec = cvec + 1
                    p = jnp.exp(acc * scale)
                    plsc.store_scatter(ebuf, [bvec, hvecs[h]], p)
                return c2

            lax.fori_loop(0, _B // _L, group, 0)
            pltpu.sync_copy(ebuf, p_out.at[pl.ds(ebase, _B)])
            pltpu.sync_copy(ebuf, dsp.at[didx], add=True)
            return carry

        lax.fori_loop(0, _CH, chunk, 0)
        plsc.subcore_barrier()
        pltpu.sync_copy(dsp.at[pl.ds(sid * _RPT, _RPT)],
                        dpart.at[cid, pl.ds(sid * _RPT, _RPT)])

    return pass1


_NEP = 4
_HPE = _H // _NEP
_DH = _HPE * _L


@functools.partial(
    pl.kernel,
    out_type=jax.ShapeDtypeStruct((_NC, _NEP, _NPAD, _DH), _f32),
    mesh=_mesh(),
    compiler_params=pltpu.CompilerParams(needs_layout_passes=False, use_tc_tiling_on_sc=False),
    scratch_types=[
        pltpu.VMEM((_B,), _i32),
        pltpu.VMEM((_B,), _i32),
        pltpu.VMEM((_B, _D1), _f32),
        pltpu.VMEM((_B, 16), _f32),
        pltpu.VMEM((_B, _DH), _f32),
        pltpu.VMEM_SHARED((_NPAD, _DH), _f32),
        pltpu.SemaphoreType.DMA,
    ],
)
def _pass2_l1(xw, src, dst, p_in, zr, m_out,
              sidx, didx, srows, pbuf, mbuf, msp, sem):
    cid = lax.axis_index("c")
    sid = lax.axis_index("s")
    wid = sid * _NC + cid
    hvecs = [jnp.full((_L,), h, _i32) for h in range(_H)]

    for half in range(_NEP):
        pltpu.sync_copy(zr, msp.at[pl.ds(sid * _RPT, _RPT)])
        plsc.subcore_barrier()

        def chunk(j, carry):
            ebase = wid * _EW + j * _B
            pltpu.sync_copy(src.at[pl.ds(ebase, _B)], sidx)
            pltpu.sync_copy(dst.at[pl.ds(ebase, _B)], didx)
            pltpu.async_copy(xw.at[sidx], srows, sem).wait()
            pltpu.sync_copy(p_in.at[pl.ds(ebase, _B)], pbuf)

            def edge(b, c2):
                bvec = jnp.full((_L,), b, _i32)
                for hl in range(_HPE):
                    h = half * _HPE + hl
                    asp = plsc.load_gather(pbuf, [bvec, hvecs[h]])
                    m = srows[b, pl.ds(h * _L, _L)] * asp
                    mbuf[b, pl.ds(hl * _L, _L)] = m
                return c2

            lax.fori_loop(0, _B, edge, 0)
            pltpu.sync_copy(mbuf, msp.at[didx], add=True)
            return carry

        lax.fori_loop(0, _CH, chunk, 0)
        plsc.subcore_barrier()
        pltpu.sync_copy(msp.at[pl.ds(sid * _RPT, _RPT)],
                        m_out.at[cid, half, pl.ds(sid * _RPT, _RPT)])
        plsc.subcore_barrier()


@functools.partial(
    pl.kernel,
    out_type=(
        jax.ShapeDtypeStruct((_EPAD, 16), _f32),
        jax.ShapeDtypeStruct((_NC, _NPAD, _C2P), _f32),
    ),
    mesh=_mesh(),
    compiler_params=pltpu.CompilerParams(needs_layout_passes=False, use_tc_tiling_on_sc=False),
    scratch_types=[
        pltpu.VMEM((_B,), _i32),
        pltpu.VMEM((_B,), _i32),
        pltpu.VMEM((_B, _D2), _f32),
        pltpu.VMEM((_B, 16), _f32),
        pltpu.VMEM((_B, 128), _f32),
        pltpu.VMEM((_B, 16), _f32),
        pltpu.VMEM((_B, _C2P), _f32),
        pltpu.VMEM_SHARED((_NPAD, _C2P), _f32),
        pltpu.SemaphoreType.DMA,
    ],
)
def _pass2_l2(xw, src, dst, p_in, invd, zr, a_out, m_out,
              sidx, didx, srows, pbuf, ivd, abuf, mbuf, msp, sem):
    cid = lax.axis_index("c")
    sid = lax.axis_index("s")
    wid = sid * _NC + cid
    hvecs = [jnp.full((_L,), h, _i32) for h in range(_H)]
    pltpu.sync_copy(zr, msp.at[pl.ds(sid * _RPT, _RPT)])
    plsc.subcore_barrier()

    def chunk(j, carry):
        ebase = wid * _EW + j * _B
        pltpu.sync_copy(src.at[pl.ds(ebase, _B)], sidx)
        pltpu.sync_copy(dst.at[pl.ds(ebase, _B)], didx)
        pltpu.async_copy(xw.at[sidx], srows, sem).wait()
        pltpu.async_copy(invd.at[didx], ivd, sem).wait()
        pltpu.sync_copy(p_in.at[pl.ds(ebase, _B)], pbuf)

        def edge(b, c2):
            bvec = jnp.full((_L,), b, _i32)
            arow = pbuf[b, :] * ivd[b, pl.ds(0, _L)]
            abuf[b, :] = arow
            macc = [jnp.zeros((_L,), _f32) for _ in range(3)]
            for h in range(_H):
                asp = plsc.load_gather(abuf, [bvec, hvecs[h]])
                for t in range(3):
                    o = h * _C2P + t * _L
                    macc[t] = macc[t] + srows[b, pl.ds(o, _L)] * asp
            for t in range(3):
                mbuf[b, pl.ds(t * _L, _L)] = macc[t]
            return c2

        lax.fori_loop(0, _B, edge, 0)
        pltpu.sync_copy(abuf, a_out.at[pl.ds(ebase, _B)])
        pltpu.sync_copy(mbuf, msp.at[didx], add=True)
        return carry

    lax.fori_loop(0, _CH, chunk, 0)
    plsc.subcore_barrier()
    pltpu.sync_copy(msp.at[pl.ds(sid * _RPT, _RPT)],
                    m_out.at[cid, pl.ds(sid * _RPT, _RPT)])


_RWI = _NPAD // _NW


@functools.partial(
    pl.kernel,
    out_type=jax.ShapeDtypeStruct((_NPAD, 128), _f32),
    mesh=_mesh(),
    compiler_params=pltpu.CompilerParams(needs_layout_passes=False, use_tc_tiling_on_sc=False),
    scratch_types=[
        pltpu.VMEM((_RWI, 16), _f32),
        pltpu.VMEM((_RWI, 16), _f32),
        pltpu.VMEM((_RWI, 128), _f32),
    ],
)
def _invd_table(dp, out, b0, b1, wide):
    cid = lax.axis_index("c")
    sid = lax.axis_index("s")
    wid = sid * _NC + cid
    base = wid * _RWI
    pltpu.sync_copy(dp.at[0, pl.ds(base, _RWI)], b0)
    pltpu.sync_copy(dp.at[1, pl.ds(base, _RWI)], b1)

    def row(r, carry):
        v = 1.0 / (b0[r, :] + b1[r, :] + 1e-16)
        for t in range(8):
            wide[r, pl.ds(t * _L, _L)] = v
        return carry

    lax.fori_loop(0, _RWI, row, 0)
    pltpu.sync_copy(wide, out.at[pl.ds(base, _RWI)])


_RB = 512


def _mm1_body(x_ref, w_ref, o_ref):
    o_ref[...] = jnp.dot(x_ref[...], w_ref[...],
                         preferred_element_type=_f32)


def _tc_mm1(x_pad, W1):
    return pl.pallas_call(
        _mm1_body,
        grid=(_NPAD // _RB,),
        in_specs=[
            pl.BlockSpec((_RB, _D1), lambda i: (i, 0)),
            pl.BlockSpec((_D1, _D1), lambda i: (0, 0)),
        ],
        out_specs=pl.BlockSpec((_RB, _D1), lambda i: (i, 0)),
        out_shape=jax.ShapeDtypeStruct((_NPAD, _D1), _f32),
    )(x_pad, W1)


def _mm2_body(m0_ref, m1_ref, d0_ref, d1_ref, r_ref, b1_ref, w2_ref, o_ref):
    m0 = jnp.concatenate([m0_ref[0, e] for e in range(_NEP)], axis=-1)
    m1 = jnp.concatenate([m1_ref[0, e] for e in range(_NEP)], axis=-1)
    den = d0_ref[0] + d1_ref[0] + 1e-16
    den128 = jnp.dot(den, r_ref[...], preferred_element_type=_f32)
    s = (m0 + m1) / den128 + b1_ref[...]
    x1 = jnp.where(s > 0.0, s, jnp.exp(s) - 1.0)
    o_ref[...] = jnp.dot(x1, w2_ref[...], preferred_element_type=_f32)


def _tc_mm2(m1part, dpart1, R, b1, W2p):
    return pl.pallas_call(
        _mm2_body,
        grid=(_NPAD // _RB,),
        in_specs=[
            pl.BlockSpec((1, _NEP, _RB, _DH), lambda i: (0, 0, i, 0)),
            pl.BlockSpec((1, _NEP, _RB, _DH), lambda i: (1, 0, i, 0)),
            pl.BlockSpec((1, _RB, 16), lambda i: (0, i, 0)),
            pl.BlockSpec((1, _RB, 16), lambda i: (1, i, 0)),
            pl.BlockSpec((16, _D1), lambda i: (0, 0)),
            pl.BlockSpec((1, _D1), lambda i: (0, 0)),
            pl.BlockSpec((_D1, _D2), lambda i: (0, 0)),
        ],
        out_specs=pl.BlockSpec((_RB, _D2), lambda i: (i, 0)),
        out_shape=jax.ShapeDtypeStruct((_NPAD, _D2), _f32),
    )(m1part, m1part, dpart1, dpart1, R, b1, W2p)


def _copy_body(x_ref, o_ref):
    o_ref[...] = x_ref[...]


def _tc_slice_rows(x, rows, rb):
    w = x.shape[1]
    return pl.pallas_call(
        _copy_body,
        grid=(rows // rb,),
        in_specs=[pl.BlockSpec((rb, w), lambda i: (i, 0))],
        out_specs=pl.BlockSpec((rb, w), lambda i: (i, 0)),
        out_shape=jax.ShapeDtypeStruct((rows, w), _f32),
    )(x)


def _sel_body(x_ref, s_ref, o_ref):
    o_ref[...] = jnp.dot(x_ref[...], s_ref[...], preferred_element_type=_f32)


def _tc_select_cols(x, S, rows, rb):
    w = x.shape[1]
    w2 = S.shape[1]
    return pl.pallas_call(
        _sel_body,
        grid=(rows // rb,),
        in_specs=[
            pl.BlockSpec((rb, w), lambda i: (i, 0)),
            pl.BlockSpec((w, w2), lambda i: (0, 0)),
        ],
        out_specs=pl.BlockSpec((rb, w2), lambda i: (i, 0)),
        out_shape=jax.ShapeDtypeStruct((rows, w2), _f32),
    )(x, S)


def _out2_body(m0_ref, m1_ref, b2_ref, s_ref, o_ref):
    r = (m0_ref[0] + m1_ref[0]) * (1.0 / _H) + b2_ref[...]
    o_ref[...] = jnp.dot(r, s_ref[...], preferred_element_type=_f32)


def _tc_out2(m2part, b2p, S48):
    rb = 400
    return pl.pallas_call(
        _out2_body,
        grid=(_N // rb,),
        in_specs=[
            pl.BlockSpec((1, rb, _C2P), lambda i: (0, i, 0)),
            pl.BlockSpec((1, rb, _C2P), lambda i: (1, i, 0)),
            pl.BlockSpec((1, _C2P), lambda i: (0, 0)),
            pl.BlockSpec((_C2P, _C2), lambda i: (0, 0)),
        ],
        out_specs=pl.BlockSpec((rb, _C2), lambda i: (i, 0)),
        out_shape=jax.ShapeDtypeStruct((_N, _C2), _f32),
    )(m2part, m2part, b2p, S48)


_pass1_l1 = _make_pass1(_D1, 1, _SCALE1)
_pass1_l2 = _make_pass1(_D2, 3, _SCALE2)


def kernel(x, edge_index, W1, b1, W2, b2):
    x = x.astype(_f32)
    ei = edge_index.astype(_i32)
    loop = jnp.arange(_N, dtype=_i32)
    npad_e = _EPAD - _E_REAL
    src = jnp.concatenate([ei[0], loop, jnp.zeros((npad_e,), _i32)])
    dst = jnp.concatenate([ei[1], loop, jnp.full((npad_e,), _N, _i32)])

    x_pad = jnp.pad(x, ((0, _NPAD - _N), (0, 0)))
    W2p = jnp.pad(W2.reshape(_D1, _H, _C2).astype(_f32),
                  ((0, 0), (0, 0), (0, _C2P - _C2))).reshape(_D1, _D2)
    b1r = b1.astype(_f32).reshape(1, _D1)
    b2pr = jnp.pad(b2.astype(_f32), (0, _C2P - _C2)).reshape(1, _C2P)

    zr16 = jnp.zeros((_RPT, 16), _f32)
    zr64 = jnp.zeros((_RPT, _DH), _f32)
    zr48 = jnp.zeros((_RPT, _C2P), _f32)

    R = (jnp.repeat(jnp.eye(8, dtype=_f32), 16, axis=1))
    R = jnp.concatenate([R, jnp.zeros((8, _D1), _f32)], axis=0)
    eye48 = jnp.eye(_C2P, dtype=_f32)
    S48 = eye48[:, :_C2]
    Sh2 = jnp.zeros((_D2, _H * _C2), _f32)
    for h in range(_H):
        Sh2 = Sh2.at[h * _C2P:h * _C2P + _C2,
                     h * _C2:(h + 1) * _C2].set(jnp.eye(_C2, dtype=_f32))
    S16 = jnp.eye(16, dtype=_f32)[:, :_H]

    xw1 = _tc_mm1(x_pad, W1.astype(_f32))
    p1, dpart1 = _pass1_l1(xw1, src, dst, zr16)
    m1part = _pass2_l1(xw1, src, dst, p1, zr64)
    if isinstance(m1part, (tuple, list)):
        m1part = m1part[0]
    xw2 = _tc_mm2(m1part, dpart1, R, b1r, W2p)
    p2, dpart2 = _pass1_l2(xw2, src, dst, zr16)
    invd2 = _invd_table(dpart2)
    a2_16, m2part = _pass2_l2(xw2, src, dst, p2, invd2, zr48)

    out2 = _tc_out2(m2part, b2pr, S48)
    h1 = _tc_slice_rows(xw1, _N, 400).reshape(_N, _H, _C1)
    h2 = _tc_select_cols(xw2, Sh2, _N, 400).reshape(_N, _H, _C2)
    a2 = _tc_select_cols(a2_16, S16, _E_REAL, 2000)
    return (out2, h1, h2, a2)


# speedup vs baseline: 10.5599x; 1.1055x over previous
import functools
import math

import jax
import jax.numpy as jnp
from jax import lax
from jax.experimental import pallas as pl
from jax.experimental.pallas import tpu as pltpu
from jax.experimental.pallas import tpu_sc as plsc

_N = 10000
_E_REAL = 330000
_H = 8
_C1 = 16
_C2 = 40
_C2P = 48
_D1 = _H * _C1
_D2 = _H * _C2P

_NC = 2
_NS = 16
_L = 16
_NW = _NC * _NS

_B = 128
_CH = 81
_EW = _B * _CH
_EPAD = _EW * _NW
_NPAD = 10016
_RPT = _NPAD // _NS

_SCALE1 = 0.25
_SCALE2 = 1.0 / math.sqrt(40.0)

_f32 = jnp.float32
_i32 = jnp.int32


def _mesh():
    return plsc.VectorSubcoreMesh(core_axis_name="c", subcore_axis_name="s")


def _make_pass1(D, nsub, scale):
    hs = nsub * _L

    @functools.partial(
        pl.kernel,
        out_type=(
            jax.ShapeDtypeStruct((_EPAD, 16), _f32),
            jax.ShapeDtypeStruct((_NC, _NPAD, 16), _f32),
        ),
        mesh=_mesh(),
        compiler_params=pltpu.CompilerParams(needs_layout_passes=False, use_tc_tiling_on_sc=False),
        scratch_types=[
            pltpu.VMEM((_B,), _i32),
            pltpu.VMEM((_B,), _i32),
            pltpu.VMEM((_B, D), _f32),
            pltpu.VMEM((_B, D), _f32),
            pltpu.VMEM((_B, 16), _f32),
            pltpu.VMEM_SHARED((_NPAD, 16), _f32),
            pltpu.SemaphoreType.DMA,
            pltpu.SemaphoreType.DMA,
        ],
    )
    def pass1(xw, src, dst, zr, p_out, dpart, sidx, didx, srows, drows,
              ebuf, dsp, sem, sem2):
        cid = lax.axis_index("c")
        sid = lax.axis_index("s")
        wid = sid * _NC + cid
        pltpu.sync_copy(zr, dsp.at[pl.ds(sid * _RPT, _RPT)])
        plsc.subcore_barrier()
        lanes = lax.iota(_i32, _L)

        def zrow(b, c2):
            ebuf[b, :] = jnp.zeros((_L,), _f32)
            return c2

        lax.fori_loop(0, _B, zrow, 0)
        hvecs = [jnp.full((_L,), h, _i32) for h in range(_H)]

        def chunk(j, carry):
            ebase = wid * _EW + j * _B
            pltpu.sync_copy(src.at[pl.ds(ebase, _B)], sidx)
            pltpu.sync_copy(dst.at[pl.ds(ebase, _B)], didx)
            cp1 = pltpu.async_copy(xw.at[sidx], srows, sem)
            cp2 = pltpu.async_copy(xw.at[didx], drows, sem2)
            cp1.wait()
            cp2.wait()

            def group(g, c2):
                bvec = g * _L + lanes
                acc = [jnp.zeros((_L,), _f32) for _ in range(_H)]
                for c in range(hs):
                    for h in range(_H):
                        cvec = jnp.full((_L,), h * hs + c, _i32)
                        sv = plsc.load_gather(srows, [bvec, cvec])
                        dv = plsc.load_gather(drows, [bvec, cvec])
                        acc[h] = acc[h] + sv * dv
                for h in range(_H):
                    pv = jnp.exp(acc[h] * scale)
                    plsc.store_scatter(ebuf, [bvec, hvecs[h]], pv)
                return c2

            lax.fori_loop(0, _B // _L, group, 0)
            pltpu.sync_copy(ebuf, p_out.at[pl.ds(ebase, _B)])
            pltpu.sync_copy(ebuf, dsp.at[didx], add=True)
            return carry

        lax.fori_loop(0, _CH, chunk, 0)
        plsc.subcore_barrier()
        pltpu.sync_copy(dsp.at[pl.ds(sid * _RPT, _RPT)],
                        dpart.at[cid, pl.ds(sid * _RPT, _RPT)])

    return pass1


@functools.partial(
    pl.kernel,
    out_type=jax.ShapeDtypeStruct((_NC, _NPAD, _D1), _f32),
    mesh=_mesh(),
    compiler_params=pltpu.CompilerParams(needs_layout_passes=False, use_tc_tiling_on_sc=False),
    scratch_types=[
        pltpu.VMEM((_B,), _i32),
        pltpu.VMEM((_B,), _i32),
        pltpu.VMEM((_B, _D1), _f32),
        pltpu.VMEM((_B, 16), _f32),
        pltpu.VMEM((_B, _D1), _f32),
        pltpu.VMEM_SHARED((_NPAD, _D1), _f32),
        pltpu.SemaphoreType.DMA,
    ],
)
def _pass2_l1(xw, src, dst, p_in, zr, m_out,
              sidx, didx, srows, pbuf, mbuf, msp, sem):
    cid = lax.axis_index("c")
    sid = lax.axis_index("s")
    wid = sid * _NC + cid
    hvecs = [jnp.full((_L,), h, _i32) for h in range(_H)]
    pltpu.sync_copy(zr, msp.at[pl.ds(sid * _RPT, _RPT)])
    plsc.subcore_barrier()

    def chunk(j, carry):
        ebase = wid * _EW + j * _B
        pltpu.sync_copy(src.at[pl.ds(ebase, _B)], sidx)
        pltpu.sync_copy(dst.at[pl.ds(ebase, _B)], didx)
        pltpu.async_copy(xw.at[sidx], srows, sem).wait()
        pltpu.sync_copy(p_in.at[pl.ds(ebase, _B)], pbuf)

        def edge(b, c2):
            bvec = jnp.full((_L,), b, _i32)
            for h in range(_H):
                asp = plsc.load_gather(pbuf, [bvec, hvecs[h]])
                m = srows[b, pl.ds(h * _L, _L)] * asp
                mbuf[b, pl.ds(h * _L, _L)] = m
            return c2

        lax.fori_loop(0, _B, edge, 0)
        pltpu.sync_copy(mbuf, msp.at[didx], add=True)
        return carry

    lax.fori_loop(0, _CH, chunk, 0)
    plsc.subcore_barrier()
    pltpu.sync_copy(msp.at[pl.ds(sid * _RPT, _RPT)],
                    m_out.at[cid, pl.ds(sid * _RPT, _RPT)])


@functools.partial(
    pl.kernel,
    out_type=(
        jax.ShapeDtypeStruct((_EPAD, 16), _f32),
        jax.ShapeDtypeStruct((_NC, _NPAD, _C2P), _f32),
    ),
    mesh=_mesh(),
    compiler_params=pltpu.CompilerParams(needs_layout_passes=False, use_tc_tiling_on_sc=False),
    scratch_types=[
        pltpu.VMEM((_B,), _i32),
        pltpu.VMEM((_B,), _i32),
        pltpu.VMEM((_B, _D2), _f32),
        pltpu.VMEM((_B, 16), _f32),
        pltpu.VMEM((_B, 128), _f32),
        pltpu.VMEM((_B, 16), _f32),
        pltpu.VMEM((_B, _C2P), _f32),
        pltpu.VMEM_SHARED((_NPAD, _C2P), _f32),
        pltpu.SemaphoreType.DMA,
        pltpu.SemaphoreType.DMA,
    ],
)
def _pass2_l2(xw, src, dst, p_in, invd, zr, a_out, m_out,
              sidx, didx, srows, pbuf, ivd, abuf, mbuf, msp, sem, sem2):
    cid = lax.axis_index("c")
    sid = lax.axis_index("s")
    wid = sid * _NC + cid
    hvecs = [jnp.full((_L,), h, _i32) for h in range(_H)]
    pltpu.sync_copy(zr, msp.at[pl.ds(sid * _RPT, _RPT)])
    plsc.subcore_barrier()

    def chunk(j, carry):
        ebase = wid * _EW + j * _B
        pltpu.sync_copy(src.at[pl.ds(ebase, _B)], sidx)
        pltpu.sync_copy(dst.at[pl.ds(ebase, _B)], didx)
        cp1 = pltpu.async_copy(xw.at[sidx], srows, sem)
        cp2 = pltpu.async_copy(invd.at[didx], ivd, sem2)
        pltpu.sync_copy(p_in.at[pl.ds(ebase, _B)], pbuf)
        cp1.wait()
        cp2.wait()

        def edge(b, c2):
            bvec = jnp.full((_L,), b, _i32)
            arow = pbuf[b, :] * ivd[b, pl.ds(0, _L)]
            abuf[b, :] = arow
            macc = [jnp.zeros((_L,), _f32) for _ in range(3)]
            for h in range(_H):
                asp = plsc.load_gather(abuf, [bvec, hvecs[h]])
                for t in range(3):
                    o = h * _C2P + t * _L
                    macc[t] = macc[t] + srows[b, pl.ds(o, _L)] * asp
            for t in range(3):
                mbuf[b, pl.ds(t * _L, _L)] = macc[t]
            return c2

        lax.fori_loop(0, _B, edge, 0)
        pltpu.sync_copy(abuf, a_out.at[pl.ds(ebase, _B)])
        pltpu.sync_copy(mbuf, msp.at[didx], add=True)
        return carry

    lax.fori_loop(0, _CH, chunk, 0)
    plsc.subcore_barrier()
    pltpu.sync_copy(msp.at[pl.ds(sid * _RPT, _RPT)],
                    m_out.at[cid, pl.ds(sid * _RPT, _RPT)])


_RWI = _NPAD // _NW


@functools.partial(
    pl.kernel,
    out_type=jax.ShapeDtypeStruct((_NPAD, 128), _f32),
    mesh=_mesh(),
    compiler_params=pltpu.CompilerParams(needs_layout_passes=False, use_tc_tiling_on_sc=False),
    scratch_types=[
        pltpu.VMEM((_RWI, 16), _f32),
        pltpu.VMEM((_RWI, 16), _f32),
        pltpu.VMEM((_RWI, 128), _f32),
    ],
)
def _invd_table(dp, out, b0, b1, wide):
    cid = lax.axis_index("c")
    sid = lax.axis_index("s")
    wid = sid * _NC + cid
    base = wid * _RWI
    pltpu.sync_copy(dp.at[0, pl.ds(base, _RWI)], b0)
    pltpu.sync_copy(dp.at[1, pl.ds(base, _RWI)], b1)

    def row(r, carry):
        v = 1.0 / (b0[r, :] + b1[r, :] + 1e-16)
        for t in range(8):
            wide[r, pl.ds(t * _L, _L)] = v
        return carry

    lax.fori_loop(0, _RWI, row, 0)
    pltpu.sync_copy(wide, out.at[pl.ds(base, _RWI)])


_RB = 2504


def _mm1_body(x_ref, w_ref, o_ref):
    o_ref[...] = jnp.dot(x_ref[...], w_ref[...],
                         preferred_element_type=_f32)


def _tc_mm1(x_pad, W1):
    return pl.pallas_call(
        _mm1_body,
        grid=(_NPAD // _RB,),
        in_specs=[
            pl.BlockSpec((_RB, _D1), lambda i: (i, 0)),
            pl.BlockSpec((_D1, _D1), lambda i: (0, 0)),
        ],
        out_specs=pl.BlockSpec((_RB, _D1), lambda i: (i, 0)),
        out_shape=jax.ShapeDtypeStruct((_NPAD, _D1), _f32),
    )(x_pad, W1)


def _mm2_body(m0_ref, m1_ref, d0_ref, d1_ref, r_ref, b1_ref, w2_ref, o_ref):
    m0 = m0_ref[0]
    m1 = m1_ref[0]
    den = d0_ref[0] + d1_ref[0] + 1e-16
    den128 = jnp.dot(den, r_ref[...], preferred_element_type=_f32)
    s = (m0 + m1) / den128 + b1_ref[...]
    x1 = jnp.where(s > 0.0, s, jnp.exp(s) - 1.0)
    o_ref[...] = jnp.dot(x1, w2_ref[...], preferred_element_type=_f32)


def _tc_mm2(m1part, dpart1, R, b1, W2p):
    return pl.pallas_call(
        _mm2_body,
        grid=(_NPAD // _RB,),
        in_specs=[
            pl.BlockSpec((1, _RB, _D1), lambda i: (0, i, 0)),
            pl.BlockSpec((1, _RB, _D1), lambda i: (1, i, 0)),
            pl.BlockSpec((1, _RB, 16), lambda i: (0, i, 0)),
            pl.BlockSpec((1, _RB, 16), lambda i: (1, i, 0)),
            pl.BlockSpec((16, _D1), lambda i: (0, 0)),
            pl.BlockSpec((1, _D1), lambda i: (0, 0)),
            pl.BlockSpec((_D1, _D2), lambda i: (0, 0)),
        ],
        out_specs=pl.BlockSpec((_RB, _D2), lambda i: (i, 0)),
        out_shape=jax.ShapeDtypeStruct((_NPAD, _D2), _f32),
    )(m1part, m1part, dpart1, dpart1, R, b1, W2p)


def _copy_body(x_ref, o_ref):
    o_ref[...] = x_ref[...]


def _tc_slice_rows(x, rows, rb):
    w = x.shape[1]
    return pl.pallas_call(
        _copy_body,
        grid=(rows // rb,),
        in_specs=[pl.BlockSpec((rb, w), lambda i: (i, 0))],
        out_specs=pl.BlockSpec((rb, w), lambda i: (i, 0)),
        out_shape=jax.ShapeDtypeStruct((rows, w), _f32),
    )(x)


def _sel_body(x_ref, s_ref, o_ref):
    o_ref[...] = jnp.dot(x_ref[...], s_ref[...], preferred_element_type=_f32)


def _tc_select_cols(x, S, rows, rb):
    w = x.shape[1]
    w2 = S.shape[1]
    return pl.pallas_call(
        _sel_body,
        grid=(rows // rb,),
        in_specs=[
            pl.BlockSpec((rb, w), lambda i: (i, 0)),
            pl.BlockSpec((w, w2), lambda i: (0, 0)),
        ],
        out_specs=pl.BlockSpec((rb, w2), lambda i: (i, 0)),
        out_shape=jax.ShapeDtypeStruct((rows, w2), _f32),
    )(x, S)


def _out2_body(m0_ref, m1_ref, b2_ref, s_ref, o_ref):
    r = (m0_ref[0] + m1_ref[0]) * (1.0 / _H) + b2_ref[...]
    o_ref[...] = jnp.dot(r, s_ref[...], preferred_element_type=_f32)


def _tc_out2(m2part, b2p, S48):
    rb = 400
    return pl.pallas_call(
        _out2_body,
        grid=(_N // rb,),
        in_specs=[
            pl.BlockSpec((1, rb, _C2P), lambda i: (0, i, 0)),
            pl.BlockSpec((1, rb, _C2P), lambda i: (1, i, 0)),
            pl.BlockSpec((1, _C2P), lambda i: (0, 0)),
            pl.BlockSpec((_C2P, _C2), lambda i: (0, 0)),
        ],
        out_specs=pl.BlockSpec((rb, _C2), lambda i: (i, 0)),
        out_shape=jax.ShapeDtypeStruct((_N, _C2), _f32),
    )(m2part, m2part, b2p, S48)


_pass1_l1 = _make_pass1(_D1, 1, _SCALE1)
_pass1_l2 = _make_pass1(_D2, 3, _SCALE2)


def kernel(x, edge_index, W1, b1, W2, b2):
    x = x.astype(_f32)
    ei = edge_index.astype(_i32)
    loop = jnp.arange(_N, dtype=_i32)
    npad_e = _EPAD - _E_REAL
    src = jnp.concatenate([ei[0], loop, jnp.zeros((npad_e,), _i32)])
    dst = jnp.concatenate([ei[1], loop, jnp.full((npad_e,), _N, _i32)])

    x_pad = jnp.pad(x, ((0, _NPAD - _N), (0, 0)))
    W2p = jnp.pad(W2.reshape(_D1, _H, _C2).astype(_f32),
                  ((0, 0), (0, 0), (0, _C2P - _C2))).reshape(_D1, _D2)
    b1r = b1.astype(_f32).reshape(1, _D1)
    b2pr = jnp.pad(b2.astype(_f32), (0, _C2P - _C2)).reshape(1, _C2P)

    zr16 = jnp.zeros((_RPT, 16), _f32)
    zr128 = jnp.zeros((_RPT, _D1), _f32)
    zr48 = jnp.zeros((_RPT, _C2P), _f32)

    R = (jnp.repeat(jnp.eye(8, dtype=_f32), 16, axis=1))
    R = jnp.concatenate([R, jnp.zeros((8, _D1), _f32)], axis=0)
    eye48 = jnp.eye(_C2P, dtype=_f32)
    S48 = eye48[:, :_C2]
    Sh2 = jnp.zeros((_D2, _H * _C2), _f32)
    for h in range(_H):
        Sh2 = Sh2.at[h * _C2P:h * _C2P + _C2,
                     h * _C2:(h + 1) * _C2].set(jnp.eye(_C2, dtype=_f32))
    S16 = jnp.eye(16, dtype=_f32)[:, :_H]

    xw1 = _tc_mm1(x_pad, W1.astype(_f32))
    p1, dpart1 = _pass1_l1(xw1, src, dst, zr16)
    m1part = _pass2_l1(xw1, src, dst, p1, zr128)
    if isinstance(m1part, (tuple, list)):
        m1part = m1part[0]
    xw2 = _tc_mm2(m1part, dpart1, R, b1r, W2p)
    p2, dpart2 = _pass1_l2(xw2, src, dst, zr16)
    invd2 = _invd_table(dpart2)
    a2_16, m2part = _pass2_l2(xw2, src, dst, p2, invd2, zr48)

    out2 = _tc_out2(m2part, b2pr, S48)
    h1 = _tc_slice_rows(xw1, _N, 400).reshape(_N, _H, _C1)
    h2 = _tc_select_cols(xw2, Sh2, _N, 400).reshape(_N, _H, _C2)
    a2 = _tc_select_cols(a2_16, S16, _E_REAL, 2000)
    return (out2, h1, h2, a2)


# speedup vs baseline: 12.4714x; 1.1810x over previous
import functools
import math

import jax
import jax.numpy as jnp
from jax import lax
from jax.experimental import pallas as pl
from jax.experimental.pallas import tpu as pltpu
from jax.experimental.pallas import tpu_sc as plsc

_N = 10000
_E_REAL = 330000
_H = 8
_C1 = 16
_C2 = 40
_C2P = 48
_D1 = _H * _C1
_D2 = _H * _C2P

_NC = 2
_NS = 16
_L = 16
_NW = _NC * _NS

_B = 128
_CH = 81
_EW = _B * _CH
_EPAD = _EW * _NW
_NPAD = 10016
_RPT = _NPAD // _NS

_SCALE1 = 0.25
_SCALE2 = 1.0 / math.sqrt(40.0)

_f32 = jnp.float32
_i32 = jnp.int32


def _mesh():
    return plsc.VectorSubcoreMesh(core_axis_name="c", subcore_axis_name="s")


def _make_pass1(D, nsub, scale, B, CH):
    hs = nsub * _L

    @functools.partial(
        pl.kernel,
        out_type=(
            jax.ShapeDtypeStruct((_EPAD, 16), _f32),
            jax.ShapeDtypeStruct((_NC, _NPAD, 16), _f32),
        ),
        mesh=_mesh(),
        compiler_params=pltpu.CompilerParams(needs_layout_passes=False, use_tc_tiling_on_sc=False),
        scratch_types=[
            pltpu.VMEM((CH, B), _i32),
            pltpu.VMEM((CH, B), _i32),
            pltpu.VMEM((B, D), _f32),
            pltpu.VMEM((B, D), _f32),
            pltpu.VMEM((B, D), _f32),
            pltpu.VMEM((B, D), _f32),
            pltpu.VMEM((B, 16), _f32),
            pltpu.SemaphoreType.DMA,
            pltpu.SemaphoreType.DMA,
            pltpu.SemaphoreType.DMA,
            pltpu.SemaphoreType.DMA,
            pltpu.VMEM_SHARED((_NPAD, 16), _f32),
        ],
    )
    def pass1(xw, src3, dst3, zr, p_out, dpart, sidx, didx,
              sr0, dr0, sr1, dr1, ebuf, ss0, ds0, ss1, ds1, dsp):
        cid = lax.axis_index("c")
        sid = lax.axis_index("s")
        wid = sid * _NC + cid
        pltpu.sync_copy(zr, dsp.at[pl.ds(sid * _RPT, _RPT)])
        plsc.subcore_barrier()
        lanes = lax.iota(_i32, _L)

        pltpu.sync_copy(src3.at[wid], sidx)
        pltpu.sync_copy(dst3.at[wid], didx)

        def zrow(b, c2):
            ebuf[b, :] = jnp.zeros((_L,), _f32)
            return c2

        lax.fori_loop(0, B, zrow, 0)
        hvecs = [jnp.full((_L,), h, _i32) for h in range(_H)]
        bufs = [(sr0, dr0, ss0, ds0), (sr1, dr1, ss1, ds1)]

        def fire(j, k):
            sr, dr, ss, ds = bufs[k]
            pltpu.async_copy(xw.at[sidx.at[j]], sr, ss)
            pltpu.async_copy(xw.at[didx.at[j]], dr, ds)

        def wait_compute(j, k):
            sr, dr, ss, ds = bufs[k]
            pltpu.make_async_copy(xw.at[sidx.at[j]], sr, ss).wait()
            pltpu.make_async_copy(xw.at[didx.at[j]], dr, ds).wait()

            def group(g, c2):
                bvec = g * _L + lanes

                def chan(c, accs):
                    cv = jnp.full((_L,), c, _i32)
                    out = []
                    for h in range(_H):
                        cvec = cv + (h * hs)
                        sv = plsc.load_gather(sr, [bvec, cvec])
                        dv = plsc.load_gather(dr, [bvec, cvec])
                        out.append(accs[h] + sv * dv)
                    return tuple(out)

                acc = lax.fori_loop(
                    0, hs, chan,
                    tuple(jnp.zeros((_L,), _f32) for _ in range(_H)))
                for h in range(_H):
                    pv = jnp.exp(acc[h] * scale)
                    plsc.store_scatter(ebuf, [bvec, hvecs[h]], pv)
                return c2

            lax.fori_loop(0, B // _L, group, 0)
            ebase = wid * _EW + j * B
            pltpu.sync_copy(ebuf, p_out.at[pl.ds(ebase, B)])
            pltpu.sync_copy(ebuf, dsp.at[didx.at[j]], add=True)

        fire(0, 0)

        def pair(jj, carry):
            j0 = 2 * jj
            fire(j0 + 1, 1)
            wait_compute(j0, 0)
            fire(j0 + 2, 0)
            wait_compute(j0 + 1, 1)
            return carry

        lax.fori_loop(0, (CH - 1) // 2, pair, 0)
        if CH % 2 == 1:
            wait_compute(CH - 1, 0)
        else:
            fire(CH - 1, 1)
            wait_compute(CH - 2, 0)
            wait_compute(CH - 1, 1)
        plsc.subcore_barrier()
        pltpu.sync_copy(dsp.at[pl.ds(sid * _RPT, _RPT)],
                        dpart.at[cid, pl.ds(sid * _RPT, _RPT)])

    return pass1


@functools.partial(
    pl.kernel,
    out_type=jax.ShapeDtypeStruct((_NC, _NPAD, _D1), _f32),
    mesh=_mesh(),
    compiler_params=pltpu.CompilerParams(needs_layout_passes=False, use_tc_tiling_on_sc=False),
    scratch_types=[
        pltpu.VMEM((_B,), _i32),
        pltpu.VMEM((_B,), _i32),
        pltpu.VMEM((_B, _D1), _f32),
        pltpu.VMEM((_B, 16), _f32),
        pltpu.VMEM((_B, _D1), _f32),
        pltpu.VMEM_SHARED((_NPAD, _D1), _f32),
        pltpu.SemaphoreType.DMA,
    ],
)
def _pass2_l1(xw, src, dst, p_in, zr, m_out,
              sidx, didx, srows, pbuf, mbuf, msp, sem):
    cid = lax.axis_index("c")
    sid = lax.axis_index("s")
    wid = sid * _NC + cid
    hvecs = [jnp.full((_L,), h, _i32) for h in range(_H)]
    pltpu.sync_copy(zr, msp.at[pl.ds(sid * _RPT, _RPT)])
    plsc.subcore_barrier()

    def chunk(j, carry):
        ebase = wid * _EW + j * _B
        pltpu.sync_copy(src.at[pl.ds(ebase, _B)], sidx)
        pltpu.sync_copy(dst.at[pl.ds(ebase, _B)], didx)
        pltpu.async_copy(xw.at[sidx], srows, sem).wait()
        pltpu.sync_copy(p_in.at[pl.ds(ebase, _B)], pbuf)

        def edge(b, c2):
            bvec = jnp.full((_L,), b, _i32)
            for h in range(_H):
                asp = plsc.load_gather(pbuf, [bvec, hvecs[h]])
                m = srows[b, pl.ds(h * _L, _L)] * asp
                mbuf[b, pl.ds(h * _L, _L)] = m
            return c2

        lax.fori_loop(0, _B, edge, 0)
        pltpu.sync_copy(mbuf, msp.at[didx], add=True)
        return carry

    lax.fori_loop(0, _CH, chunk, 0)
    plsc.subcore_barrier()
    pltpu.sync_copy(msp.at[pl.ds(sid * _RPT, _RPT)],
                    m_out.at[cid, pl.ds(sid * _RPT, _RPT)])


@functools.partial(
    pl.kernel,
    out_type=(
        jax.ShapeDtypeStruct((_EPAD, 16), _f32),
        jax.ShapeDtypeStruct((_NC, _NPAD, _C2P), _f32),
    ),
    mesh=_mesh(),
    compiler_params=pltpu.CompilerParams(needs_layout_passes=False, use_tc_tiling_on_sc=False),
    scratch_types=[
        pltpu.VMEM((_B,), _i32),
        pltpu.VMEM((_B,), _i32),
        pltpu.VMEM((_B, _D2), _f32),
        pltpu.VMEM((_B, 16), _f32),
        pltpu.VMEM((_B, 128), _f32),
        pltpu.VMEM((_B, 16), _f32),
        pltpu.VMEM((_B, _C2P), _f32),
        pltpu.VMEM_SHARED((_NPAD, _C2P), _f32),
        pltpu.SemaphoreType.DMA,
        pltpu.SemaphoreType.DMA,
    ],
)
def _pass2_l2(xw, src, dst, p_in, invd, zr, a_out, m_out,
              sidx, didx, srows, pbuf, ivd, abuf, mbuf, msp, sem, sem2):
    cid = lax.axis_index("c")
    sid = lax.axis_index("s")
    wid = sid * _NC + cid
    hvecs = [jnp.full((_L,), h, _i32) for h in range(_H)]
    pltpu.sync_copy(zr, msp.at[pl.ds(sid * _RPT, _RPT)])
    plsc.subcore_barrier()

    def chunk(j, carry):
        ebase = wid * _EW + j * _B
        pltpu.sync_copy(src.at[pl.ds(ebase, _B)], sidx)
        pltpu.sync_copy(dst.at[pl.ds(ebase, _B)], didx)
        cp1 = pltpu.async_copy(xw.at[sidx], srows, sem)
        cp2 = pltpu.async_copy(invd.at[didx], ivd, sem2)
        pltpu.sync_copy(p_in.at[pl.ds(ebase, _B)], pbuf)
        cp1.wait()
        cp2.wait()

        def edge(b, c2):
            bvec = jnp.full((_L,), b, _i32)
            arow = pbuf[b, :] * ivd[b, pl.ds(0, _L)]
            abuf[b, :] = arow
            macc = [jnp.zeros((_L,), _f32) for _ in range(3)]
            for h in range(_H):
                asp = plsc.load_gather(abuf, [bvec, hvecs[h]])
                for t in range(3):
                    o = h * _C2P + t * _L
                    macc[t] = macc[t] + srows[b, pl.ds(o, _L)] * asp
            for t in range(3):
                mbuf[b, pl.ds(t * _L, _L)] = macc[t]
            return c2

        lax.fori_loop(0, _B, edge, 0)
        pltpu.sync_copy(abuf, a_out.at[pl.ds(ebase, _B)])
        pltpu.sync_copy(mbuf, msp.at[didx], add=True)
        return carry

    lax.fori_loop(0, _CH, chunk, 0)
    plsc.subcore_barrier()
    pltpu.sync_copy(msp.at[pl.ds(sid * _RPT, _RPT)],
                    m_out.at[cid, pl.ds(sid * _RPT, _RPT)])


_RWI = _NPAD // _NW


@functools.partial(
    pl.kernel,
    out_type=jax.ShapeDtypeStruct((_NPAD, 128), _f32),
    mesh=_mesh(),
    compiler_params=pltpu.CompilerParams(needs_layout_passes=False, use_tc_tiling_on_sc=False),
    scratch_types=[
        pltpu.VMEM((_RWI, 16), _f32),
        pltpu.VMEM((_RWI, 16), _f32),
        pltpu.VMEM((_RWI, 128), _f32),
    ],
)
def _invd_table(dp, out, b0, b1, wide):
    cid = lax.axis_index("c")
    sid = lax.axis_index("s")
    wid = sid * _NC + cid
    base = wid * _RWI
    pltpu.sync_copy(dp.at[0, pl.ds(base, _RWI)], b0)
    pltpu.sync_copy(dp.at[1, pl.ds(base, _RWI)], b1)

    def row(r, carry):
        v = 1.0 / (b0[r, :] + b1[r, :] + 1e-16)
        for t in range(8):
            wide[r, pl.ds(t * _L, _L)] = v
        return carry

    lax.fori_loop(0, _RWI, row, 0)
    pltpu.sync_copy(wide, out.at[pl.ds(base, _RWI)])


_RB = 2504


def _mm1_body(x_ref, w_ref, o_ref):
    o_ref[...] = jnp.dot(x_ref[...], w_ref[...],
                         preferred_element_type=_f32)


def _tc_mm1(x_pad, W1):
    return pl.pallas_call(
        _mm1_body,
        grid=(_NPAD // _RB,),
        in_specs=[
            pl.BlockSpec((_RB, _D1), lambda i: (i, 0)),
            pl.BlockSpec((_D1, _D1), lambda i: (0, 0)),
        ],
        out_specs=pl.BlockSpec((_RB, _D1), lambda i: (i, 0)),
        out_shape=jax.ShapeDtypeStruct((_NPAD, _D1), _f32),
    )(x_pad, W1)


def _mm2_body(m0_ref, m1_ref, d0_ref, d1_ref, r_ref, b1_ref, w2_ref, o_ref):
    m0 = m0_ref[0]
    m1 = m1_ref[0]
    den = d0_ref[0] + d1_ref[0] + 1e-16
    den128 = jnp.dot(den, r_ref[...], preferred_element_type=_f32)
    s = (m0 + m1) / den128 + b1_ref[...]
    x1 = jnp.where(s > 0.0, s, jnp.exp(s) - 1.0)
    o_ref[...] = jnp.dot(x1, w2_ref[...], preferred_element_type=_f32)


def _tc_mm2(m1part, dpart1, R, b1, W2p):
    return pl.pallas_call(
        _mm2_body,
        grid=(_NPAD // _RB,),
        in_specs=[
            pl.BlockSpec((1, _RB, _D1), lambda i: (0, i, 0)),
            pl.BlockSpec((1, _RB, _D1), lambda i: (1, i, 0)),
            pl.BlockSpec((1, _RB, 16), lambda i: (0, i, 0)),
            pl.BlockSpec((1, _RB, 16), lambda i: (1, i, 0)),
            pl.BlockSpec((16, _D1), lambda i: (0, 0)),
            pl.BlockSpec((1, _D1), lambda i: (0, 0)),
            pl.BlockSpec((_D1, _D2), lambda i: (0, 0)),
        ],
        out_specs=pl.BlockSpec((_RB, _D2), lambda i: (i, 0)),
        out_shape=jax.ShapeDtypeStruct((_NPAD, _D2), _f32),
    )(m1part, m1part, dpart1, dpart1, R, b1, W2p)


def _copy_body(x_ref, o_ref):
    o_ref[...] = x_ref[...]


def _tc_slice_rows(x, rows, rb):
    w = x.shape[1]
    return pl.pallas_call(
        _copy_body,
        grid=(rows // rb,),
        in_specs=[pl.BlockSpec((rb, w), lambda i: (i, 0))],
        out_specs=pl.BlockSpec((rb, w), lambda i: (i, 0)),
        out_shape=jax.ShapeDtypeStruct((rows, w), _f32),
    )(x)


def _sel_body(x_ref, s_ref, o_ref):
    o_ref[...] = jnp.dot(x_ref[...], s_ref[...], preferred_element_type=_f32)


def _tc_select_cols(x, S, rows, rb):
    w = x.shape[1]
    w2 = S.shape[1]
    return pl.pallas_call(
        _sel_body,
        grid=(rows // rb,),
        in_specs=[
            pl.BlockSpec((rb, w), lambda i: (i, 0)),
            pl.BlockSpec((w, w2), lambda i: (0, 0)),
        ],
        out_specs=pl.BlockSpec((rb, w2), lambda i: (i, 0)),
        out_shape=jax.ShapeDtypeStruct((rows, w2), _f32),
    )(x, S)


def _out2_body(m0_ref, m1_ref, b2_ref, s_ref, o_ref):
    r = (m0_ref[0] + m1_ref[0]) * (1.0 / _H) + b2_ref[...]
    o_ref[...] = jnp.dot(r, s_ref[...], preferred_element_type=_f32)


def _tc_out2(m2part, b2p, S48):
    rb = 400
    return pl.pallas_call(
        _out2_body,
        grid=(_N // rb,),
        in_specs=[
            pl.BlockSpec((1, rb, _C2P), lambda i: (0, i, 0)),
            pl.BlockSpec((1, rb, _C2P), lambda i: (1, i, 0)),
            pl.BlockSpec((1, _C2P), lambda i: (0, 0)),
            pl.BlockSpec((_C2P, _C2), lambda i: (0, 0)),
        ],
        out_specs=pl.BlockSpec((rb, _C2), lambda i: (i, 0)),
        out_shape=jax.ShapeDtypeStruct((_N, _C2), _f32),
    )(m2part, m2part, b2p, S48)


_B1, _CH1 = 128, 81
_B2, _CH2 = 48, 216
_pass1_l1 = _make_pass1(_D1, 1, _SCALE1, _B1, _CH1)
_pass1_l2 = _make_pass1(_D2, 3, _SCALE2, _B2, _CH2)


def kernel(x, edge_index, W1, b1, W2, b2):
    x = x.astype(_f32)
    ei = edge_index.astype(_i32)
    loop = jnp.arange(_N, dtype=_i32)
    npad_e = _EPAD - _E_REAL
    src = jnp.concatenate([ei[0], loop, jnp.zeros((npad_e,), _i32)])
    dst = jnp.concatenate([ei[1], loop, jnp.full((npad_e,), _N, _i32)])

    x_pad = jnp.pad(x, ((0, _NPAD - _N), (0, 0)))
    W2p = jnp.pad(W2.reshape(_D1, _H, _C2).astype(_f32),
                  ((0, 0), (0, 0), (0, _C2P - _C2))).reshape(_D1, _D2)
    b1r = b1.astype(_f32).reshape(1, _D1)
    b2pr = jnp.pad(b2.astype(_f32), (0, _C2P - _C2)).reshape(1, _C2P)

    zr16 = jnp.zeros((_RPT, 16), _f32)
    zr128 = jnp.zeros((_RPT, _D1), _f32)
    zr48 = jnp.zeros((_RPT, _C2P), _f32)

    R = (jnp.repeat(jnp.eye(8, dtype=_f32), 16, axis=1))
    R = jnp.concatenate([R, jnp.zeros((8, _D1), _f32)], axis=0)
    eye48 = jnp.eye(_C2P, dtype=_f32)
    S48 = eye48[:, :_C2]
    Sh2 = jnp.zeros((_D2, _H * _C2), _f32)
    for h in range(_H):
        Sh2 = Sh2.at[h * _C2P:h * _C2P + _C2,
                     h * _C2:(h + 1) * _C2].set(jnp.eye(_C2, dtype=_f32))
    S16 = jnp.eye(16, dtype=_f32)[:, :_H]

    xw1 = _tc_mm1(x_pad, W1.astype(_f32))
    src3a = src.reshape(_NW, _CH1, _B1)
    dst3a = dst.reshape(_NW, _CH1, _B1)
    src3b = src.reshape(_NW, _CH2, _B2)
    dst3b = dst.reshape(_NW, _CH2, _B2)
    p1, dpart1 = _pass1_l1(xw1, src3a, dst3a, zr16)
    m1part = _pass2_l1(xw1, src, dst, p1, zr128)
    if isinstance(m1part, (tuple, list)):
        m1part = m1part[0]
    xw2 = _tc_mm2(m1part, dpart1, R, b1r, W2p)
    p2, dpart2 = _pass1_l2(xw2, src3b, dst3b, zr16)
    invd2 = _invd_table(dpart2)
    a2_16, m2part = _pass2_l2(xw2, src, dst, p2, invd2, zr48)

    out2 = _tc_out2(m2part, b2pr, S48)
    h1 = _tc_slice_rows(xw1, _N, 400).reshape(_N, _H, _C1)
    h2 = _tc_select_cols(xw2, Sh2, _N, 400).reshape(_N, _H, _C2)
    a2 = _tc_select_cols(a2_16, S16, _E_REAL, 2000)
    return (out2, h1, h2, a2)


# speedup vs baseline: 31.9910x; 2.5652x over previous
import functools
import math

import jax
import jax.numpy as jnp
from jax import lax
from jax.experimental import pallas as pl
from jax.experimental.pallas import tpu as pltpu
from jax.experimental.pallas import tpu_sc as plsc

_N = 10000
_E_REAL = 330000
_H = 8
_C1 = 16
_C2 = 40
_C2P = 48
_D1 = _H * _C1
_D2 = _H * _C2P

_NC = 2
_NS = 16
_L = 16
_NW = _NC * _NS

_B = 128
_CH = 81
_EW = _B * _CH
_EPAD = _EW * _NW
_NPAD = 10016
_RPT = _NPAD // _NS

_SCALE1 = 0.25
_SCALE2 = 1.0 / math.sqrt(40.0)

_f32 = jnp.float32
_i32 = jnp.int32


def _mesh():
    return plsc.VectorSubcoreMesh(core_axis_name="c", subcore_axis_name="s")


def _make_pass1(D, nsub, scale, B, CH):
    hs = nsub * _L

    @functools.partial(
        pl.kernel,
        out_type=(
            jax.ShapeDtypeStruct((_EPAD, 16), _f32),
            jax.ShapeDtypeStruct((_NC, _NPAD, 16), _f32),
        ),
        mesh=_mesh(),
        compiler_params=pltpu.CompilerParams(needs_layout_passes=False, use_tc_tiling_on_sc=False),
        scratch_types=[
            pltpu.VMEM((CH, B), _i32),
            pltpu.VMEM((CH, B), _i32),
            pltpu.VMEM((B, D), _f32),
            pltpu.VMEM((B, D), _f32),
            pltpu.VMEM((B, D), _f32),
            pltpu.VMEM((B, D), _f32),
            pltpu.VMEM((B, 16), _f32),
            pltpu.SemaphoreType.DMA,
            pltpu.SemaphoreType.DMA,
            pltpu.SemaphoreType.DMA,
            pltpu.SemaphoreType.DMA,
            pltpu.VMEM_SHARED((_NPAD, 16), _f32),
        ],
    )
    def pass1(xw, src3, dst3, zr, p_out, dpart, sidx, didx,
              sr0, dr0, sr1, dr1, ebuf, ss0, ds0, ss1, ds1, dsp):
        cid = lax.axis_index("c")
        sid = lax.axis_index("s")
        wid = sid * _NC + cid
        pltpu.sync_copy(zr, dsp.at[pl.ds(sid * _RPT, _RPT)])
        plsc.subcore_barrier()
        lanes = lax.iota(_i32, _L)

        pltpu.sync_copy(src3.at[wid], sidx)
        pltpu.sync_copy(dst3.at[wid], didx)

        def zrow(b, c2):
            ebuf[b, :] = jnp.zeros((_L,), _f32)
            return c2

        lax.fori_loop(0, B, zrow, 0)
        hvecs = [jnp.full((_L,), h, _i32) for h in range(_H)]
        bufs = [(sr0, dr0, ss0, ds0), (sr1, dr1, ss1, ds1)]

        def fire(j, k):
            sr, dr, ss, ds = bufs[k]
            pltpu.async_copy(xw.at[sidx.at[j]], sr, ss)
            pltpu.async_copy(xw.at[didx.at[j]], dr, ds)

        def wait_compute(j, k):
            sr, dr, ss, ds = bufs[k]
            pltpu.make_async_copy(xw.at[sidx.at[j]], sr, ss).wait()
            pltpu.make_async_copy(xw.at[didx.at[j]], dr, ds).wait()

            def group(g, c2):
                bvec = g * _L + lanes

                def chan(c, carry):
                    rot = carry[0]
                    accs = carry[1:]
                    out = []
                    for h in range(_H):
                        cvec = rot + (h * hs)
                        sv = plsc.load_gather(sr, [bvec, cvec])
                        dv = plsc.load_gather(dr, [bvec, cvec])
                        out.append(accs[h] + sv * dv)
                    rot2 = rot + 1
                    rot2 = jnp.where(rot2 >= hs, rot2 - hs, rot2)
                    return (rot2, *out)

                res = lax.fori_loop(
                    0, hs, chan,
                    (lanes,) + tuple(jnp.zeros((_L,), _f32)
                                     for _ in range(_H)))
                for h in range(_H):
                    pv = jnp.exp(res[1 + h] * scale)
                    plsc.store_scatter(ebuf, [bvec, hvecs[h]], pv)
                return c2

            lax.fori_loop(0, B // _L, group, 0)
            ebase = wid * _EW + j * B
            pltpu.sync_copy(ebuf, p_out.at[pl.ds(ebase, B)])
            pltpu.sync_copy(ebuf, dsp.at[didx.at[j]], add=True)

        fire(0, 0)

        def pair(jj, carry):
            j0 = 2 * jj
            fire(j0 + 1, 1)
            wait_compute(j0, 0)
            fire(j0 + 2, 0)
            wait_compute(j0 + 1, 1)
            return carry

        lax.fori_loop(0, (CH - 1) // 2, pair, 0)
        if CH % 2 == 1:
            wait_compute(CH - 1, 0)
        else:
            fire(CH - 1, 1)
            wait_compute(CH - 2, 0)
            wait_compute(CH - 1, 1)
        plsc.subcore_barrier()
        pltpu.sync_copy(dsp.at[pl.ds(sid * _RPT, _RPT)],
                        dpart.at[cid, pl.ds(sid * _RPT, _RPT)])

    return pass1


@functools.partial(
    pl.kernel,
    out_type=jax.ShapeDtypeStruct((_NC, _NPAD, _D1), _f32),
    mesh=_mesh(),
    compiler_params=pltpu.CompilerParams(needs_layout_passes=False, use_tc_tiling_on_sc=False),
    scratch_types=[
        pltpu.VMEM((_B,), _i32),
        pltpu.VMEM((_B,), _i32),
        pltpu.VMEM((_B, _D1), _f32),
        pltpu.VMEM((_B, 16), _f32),
        pltpu.VMEM((_B, _D1), _f32),
        pltpu.VMEM_SHARED((_NPAD, _D1), _f32),
        pltpu.SemaphoreType.DMA,
    ],
)
def _pass2_l1(xw, src, dst, p_in, zr, m_out,
              sidx, didx, srows, pbuf, mbuf, msp, sem):
    cid = lax.axis_index("c")
    sid = lax.axis_index("s")
    wid = sid * _NC + cid
    hvecs = [jnp.full((_L,), h, _i32) for h in range(_H)]
    pltpu.sync_copy(zr, msp.at[pl.ds(sid * _RPT, _RPT)])
    plsc.subcore_barrier()

    def chunk(j, carry):
        ebase = wid * _EW + j * _B
        pltpu.sync_copy(src.at[pl.ds(ebase, _B)], sidx)
        pltpu.sync_copy(dst.at[pl.ds(ebase, _B)], didx)
        pltpu.async_copy(xw.at[sidx], srows, sem).wait()
        pltpu.sync_copy(p_in.at[pl.ds(ebase, _B)], pbuf)

        def edge(b, c2):
            bvec = jnp.full((_L,), b, _i32)
            for h in range(_H):
                asp = plsc.load_gather(pbuf, [bvec, hvecs[h]])
                m = srows[b, pl.ds(h * _L, _L)] * asp
                mbuf[b, pl.ds(h * _L, _L)] = m
            return c2

        lax.fori_loop(0, _B, edge, 0)
        pltpu.sync_copy(mbuf, msp.at[didx], add=True)
        return carry

    lax.fori_loop(0, _CH, chunk, 0)
    plsc.subcore_barrier()
    pltpu.sync_copy(msp.at[pl.ds(sid * _RPT, _RPT)],
                    m_out.at[cid, pl.ds(sid * _RPT, _RPT)])


@functools.partial(
    pl.kernel,
    out_type=(
        jax.ShapeDtypeStruct((_EPAD, 16), _f32),
        jax.ShapeDtypeStruct((_NC, _NPAD, _C2P), _f32),
    ),
    mesh=_mesh(),
    compiler_params=pltpu.CompilerParams(needs_layout_passes=False, use_tc_tiling_on_sc=False),
    scratch_types=[
        pltpu.VMEM((_B,), _i32),
        pltpu.VMEM((_B,), _i32),
        pltpu.VMEM((_B, _D2), _f32),
        pltpu.VMEM((_B, 16), _f32),
        pltpu.VMEM((_B, 128), _f32),
        pltpu.VMEM((_B, 16), _f32),
        pltpu.VMEM((_B, _C2P), _f32),
        pltpu.VMEM_SHARED((_NPAD, _C2P), _f32),
        pltpu.SemaphoreType.DMA,
        pltpu.SemaphoreType.DMA,
    ],
)
def _pass2_l2(xw, src, dst, p_in, invd, zr, a_out, m_out,
              sidx, didx, srows, pbuf, ivd, abuf, mbuf, msp, sem, sem2):
    cid = lax.axis_index("c")
    sid = lax.axis_index("s")
    wid = sid * _NC + cid
    hvecs = [jnp.full((_L,), h, _i32) for h in range(_H)]
    pltpu.sync_copy(zr, msp.at[pl.ds(sid * _RPT, _RPT)])
    plsc.subcore_barrier()

    def chunk(j, carry):
        ebase = wid * _EW + j * _B
        pltpu.sync_copy(src.at[pl.ds(ebase, _B)], sidx)
        pltpu.sync_copy(dst.at[pl.ds(ebase, _B)], didx)
        cp1 = pltpu.async_copy(xw.at[sidx], srows, sem)
        cp2 = pltpu.async_copy(invd.at[didx], ivd, sem2)
        pltpu.sync_copy(p_in.at[pl.ds(ebase, _B)], pbuf)
        cp1.wait()
        cp2.wait()

        def edge(b, c2):
            bvec = jnp.full((_L,), b, _i32)
            arow = pbuf[b, :] * ivd[b, pl.ds(0, _L)]
            abuf[b, :] = arow
            macc = [jnp.zeros((_L,), _f32) for _ in range(3)]
            for h in range(_H):
                asp = plsc.load_gather(abuf, [bvec, hvecs[h]])
                for t in range(3):
                    o = h * _C2P + t * _L
                    macc[t] = macc[t] + srows[b, pl.ds(o, _L)] * asp
            for t in range(3):
                mbuf[b, pl.ds(t * _L, _L)] = macc[t]
            return c2

        lax.fori_loop(0, _B, edge, 0)
        pltpu.sync_copy(abuf, a_out.at[pl.ds(ebase, _B)])
        pltpu.sync_copy(mbuf, msp.at[didx], add=True)
        return carry

    lax.fori_loop(0, _CH, chunk, 0)
    plsc.subcore_barrier()
    pltpu.sync_copy(msp.at[pl.ds(sid * _RPT, _RPT)],
                    m_out.at[cid, pl.ds(sid * _RPT, _RPT)])


_RWI = _NPAD // _NW


@functools.partial(
    pl.kernel,
    out_type=jax.ShapeDtypeStruct((_NPAD, 128), _f32),
    mesh=_mesh(),
    compiler_params=pltpu.CompilerParams(needs_layout_passes=False, use_tc_tiling_on_sc=False),
    scratch_types=[
        pltpu.VMEM((_RWI, 16), _f32),
        pltpu.VMEM((_RWI, 16), _f32),
        pltpu.VMEM((_RWI, 128), _f32),
    ],
)
def _invd_table(dp, out, b0, b1, wide):
    cid = lax.axis_index("c")
    sid = lax.axis_index("s")
    wid = sid * _NC + cid
    base = wid * _RWI
    pltpu.sync_copy(dp.at[0, pl.ds(base, _RWI)], b0)
    pltpu.sync_copy(dp.at[1, pl.ds(base, _RWI)], b1)

    def row(r, carry):
        v = 1.0 / (b0[r, :] + b1[r, :] + 1e-16)
        for t in range(8):
            wide[r, pl.ds(t * _L, _L)] = v
        return carry

    lax.fori_loop(0, _RWI, row, 0)
    pltpu.sync_copy(wide, out.at[pl.ds(base, _RWI)])


_RB = 2504


def _mm1_body(x_ref, w_ref, o_ref):
    o_ref[...] = jnp.dot(x_ref[...], w_ref[...],
                         preferred_element_type=_f32)


def _tc_mm1(x_pad, W1):
    return pl.pallas_call(
        _mm1_body,
        grid=(_NPAD // _RB,),
        in_specs=[
            pl.BlockSpec((_RB, _D1), lambda i: (i, 0)),
            pl.BlockSpec((_D1, _D1), lambda i: (0, 0)),
        ],
        out_specs=pl.BlockSpec((_RB, _D1), lambda i: (i, 0)),
        out_shape=jax.ShapeDtypeStruct((_NPAD, _D1), _f32),
    )(x_pad, W1)


def _mm2_body(m0_ref, m1_ref, d0_ref, d1_ref, r_ref, b1_ref, w2_ref, o_ref):
    m0 = m0_ref[0]
    m1 = m1_ref[0]
    den = d0_ref[0] + d1_ref[0] + 1e-16
    den128 = jnp.dot(den, r_ref[...], preferred_element_type=_f32)
    s = (m0 + m1) / den128 + b1_ref[...]
    x1 = jnp.where(s > 0.0, s, jnp.exp(s) - 1.0)
    o_ref[...] = jnp.dot(x1, w2_ref[...], preferred_element_type=_f32)


def _tc_mm2(m1part, dpart1, R, b1, W2p):
    return pl.pallas_call(
        _mm2_body,
        grid=(_NPAD // _RB,),
        in_specs=[
            pl.BlockSpec((1, _RB, _D1), lambda i: (0, i, 0)),
            pl.BlockSpec((1, _RB, _D1), lambda i: (1, i, 0)),
            pl.BlockSpec((1, _RB, 16), lambda i: (0, i, 0)),
            pl.BlockSpec((1, _RB, 16), lambda i: (1, i, 0)),
            pl.BlockSpec((16, _D1), lambda i: (0, 0)),
            pl.BlockSpec((1, _D1), lambda i: (0, 0)),
            pl.BlockSpec((_D1, _D2), lambda i: (0, 0)),
        ],
        out_specs=pl.BlockSpec((_RB, _D2), lambda i: (i, 0)),
        out_shape=jax.ShapeDtypeStruct((_NPAD, _D2), _f32),
    )(m1part, m1part, dpart1, dpart1, R, b1, W2p)


def _copy_body(x_ref, o_ref):
    o_ref[...] = x_ref[...]


def _tc_slice_rows(x, rows, rb):
    w = x.shape[1]
    return pl.pallas_call(
        _copy_body,
        grid=(rows // rb,),
        in_specs=[pl.BlockSpec((rb, w), lambda i: (i, 0))],
        out_specs=pl.BlockSpec((rb, w), lambda i: (i, 0)),
        out_shape=jax.ShapeDtypeStruct((rows, w), _f32),
    )(x)


def _sel_body(x_ref, s_ref, o_ref):
    o_ref[...] = jnp.dot(x_ref[...], s_ref[...], preferred_element_type=_f32)


def _tc_select_cols(x, S, rows, rb):
    w = x.shape[1]
    w2 = S.shape[1]
    return pl.pallas_call(
        _sel_body,
        grid=(rows // rb,),
        in_specs=[
            pl.BlockSpec((rb, w), lambda i: (i, 0)),
            pl.BlockSpec((w, w2), lambda i: (0, 0)),
        ],
        out_specs=pl.BlockSpec((rb, w2), lambda i: (i, 0)),
        out_shape=jax.ShapeDtypeStruct((rows, w2), _f32),
    )(x, S)


def _out2_body(m0_ref, m1_ref, b2_ref, s_ref, o_ref):
    r = (m0_ref[0] + m1_ref[0]) * (1.0 / _H) + b2_ref[...]
    o_ref[...] = jnp.dot(r, s_ref[...], preferred_element_type=_f32)


def _tc_out2(m2part, b2p, S48):
    rb = 400
    return pl.pallas_call(
        _out2_body,
        grid=(_N // rb,),
        in_specs=[
            pl.BlockSpec((1, rb, _C2P), lambda i: (0, i, 0)),
            pl.BlockSpec((1, rb, _C2P), lambda i: (1, i, 0)),
            pl.BlockSpec((1, _C2P), lambda i: (0, 0)),
            pl.BlockSpec((_C2P, _C2), lambda i: (0, 0)),
        ],
        out_specs=pl.BlockSpec((rb, _C2), lambda i: (i, 0)),
        out_shape=jax.ShapeDtypeStruct((_N, _C2), _f32),
    )(m2part, m2part, b2p, S48)


_B1, _CH1 = 128, 81
_B2, _CH2 = 48, 216
_pass1_l1 = _make_pass1(_D1, 1, _SCALE1, _B1, _CH1)
_pass1_l2 = _make_pass1(_D2, 3, _SCALE2, _B2, _CH2)


def kernel(x, edge_index, W1, b1, W2, b2):
    x = x.astype(_f32)
    ei = edge_index.astype(_i32)
    loop = jnp.arange(_N, dtype=_i32)
    npad_e = _EPAD - _E_REAL
    src = jnp.concatenate([ei[0], loop, jnp.zeros((npad_e,), _i32)])
    dst = jnp.concatenate([ei[1], loop, jnp.full((npad_e,), _N, _i32)])

    x_pad = jnp.pad(x, ((0, _NPAD - _N), (0, 0)))
    W2p = jnp.pad(W2.reshape(_D1, _H, _C2).astype(_f32),
                  ((0, 0), (0, 0), (0, _C2P - _C2))).reshape(_D1, _D2)
    b1r = b1.astype(_f32).reshape(1, _D1)
    b2pr = jnp.pad(b2.astype(_f32), (0, _C2P - _C2)).reshape(1, _C2P)

    zr16 = jnp.zeros((_RPT, 16), _f32)
    zr128 = jnp.zeros((_RPT, _D1), _f32)
    zr48 = jnp.zeros((_RPT, _C2P), _f32)

    R = (jnp.repeat(jnp.eye(8, dtype=_f32), 16, axis=1))
    R = jnp.concatenate([R, jnp.zeros((8, _D1), _f32)], axis=0)
    eye48 = jnp.eye(_C2P, dtype=_f32)
    S48 = eye48[:, :_C2]
    Sh2 = jnp.zeros((_D2, _H * _C2), _f32)
    for h in range(_H):
        Sh2 = Sh2.at[h * _C2P:h * _C2P + _C2,
                     h * _C2:(h + 1) * _C2].set(jnp.eye(_C2, dtype=_f32))
    S16 = jnp.eye(16, dtype=_f32)[:, :_H]

    xw1 = _tc_mm1(x_pad, W1.astype(_f32))
    src3a = src.reshape(_NW, _CH1, _B1)
    dst3a = dst.reshape(_NW, _CH1, _B1)
    src3b = src.reshape(_NW, _CH2, _B2)
    dst3b = dst.reshape(_NW, _CH2, _B2)
    p1, dpart1 = _pass1_l1(xw1, src3a, dst3a, zr16)
    m1part = _pass2_l1(xw1, src, dst, p1, zr128)
    if isinstance(m1part, (tuple, list)):
        m1part = m1part[0]
    xw2 = _tc_mm2(m1part, dpart1, R, b1r, W2p)
    p2, dpart2 = _pass1_l2(xw2, src3b, dst3b, zr16)
    invd2 = _invd_table(dpart2)
    a2_16, m2part = _pass2_l2(xw2, src, dst, p2, invd2, zr48)

    out2 = _tc_out2(m2part, b2pr, S48)
    h1 = _tc_slice_rows(xw1, _N, 400).reshape(_N, _H, _C1)
    h2 = _tc_select_cols(xw2, Sh2, _N, 400).reshape(_N, _H, _C2)
    a2 = _tc_select_cols(a2_16, S16, _E_REAL, 2000)
    return (out2, h1, h2, a2)


# speedup vs baseline: 36.8446x; 1.1517x over previous
import functools
import math

import jax
import jax.numpy as jnp
from jax import lax
from jax.experimental import pallas as pl
from jax.experimental.pallas import tpu as pltpu
from jax.experimental.pallas import tpu_sc as plsc

_N = 10000
_E_REAL = 330000
_H = 8
_C1 = 16
_C2 = 40
_C2P = 48
_D1 = _H * _C1
_D2 = _H * _C2P

_NC = 2
_NS = 16
_L = 16
_NW = _NC * _NS

_B = 128
_CH = 81
_EW = _B * _CH
_EPAD = _EW * _NW
_NPAD = 10016
_RPT = _NPAD // _NS

_SCALE1 = 0.25
_SCALE2 = 1.0 / math.sqrt(40.0)

_f32 = jnp.float32
_i32 = jnp.int32


def _mesh():
    return plsc.VectorSubcoreMesh(core_axis_name="c", subcore_axis_name="s")


def _take16(v, idx):
    dn = lax.GatherDimensionNumbers(offset_dims=(), collapsed_slice_dims=(0,),
                                    start_index_map=(0,))
    return lax.gather(v, idx[:, None], dn, slice_sizes=(1,),
                      mode=lax.GatherScatterMode.PROMISE_IN_BOUNDS)


def _make_pass1(D, nsub, scale, B, CH):
    hs = nsub * _L

    @functools.partial(
        pl.kernel,
        out_type=(
            jax.ShapeDtypeStruct((_EPAD, 16), _f32),
            jax.ShapeDtypeStruct((_NC, _NPAD, 16), _f32),
        ),
        mesh=_mesh(),
        compiler_params=pltpu.CompilerParams(needs_layout_passes=False, use_tc_tiling_on_sc=False),
        scratch_types=[
            pltpu.VMEM((CH, B), _i32),
            pltpu.VMEM((CH, B), _i32),
            pltpu.VMEM((B, D), _f32),
            pltpu.VMEM((B, D), _f32),
            pltpu.VMEM((B, D), _f32),
            pltpu.VMEM((B, D), _f32),
            pltpu.VMEM((B, 16), _f32),
            pltpu.SemaphoreType.DMA,
            pltpu.SemaphoreType.DMA,
            pltpu.SemaphoreType.DMA,
            pltpu.SemaphoreType.DMA,
            pltpu.VMEM_SHARED((_NPAD, 16), _f32),
        ],
    )
    def pass1(xw, src3, dst3, zr, p_out, dpart, sidx, didx,
              sr0, dr0, sr1, dr1, ebuf, ss0, ds0, ss1, ds1, dsp):
        cid = lax.axis_index("c")
        sid = lax.axis_index("s")
        wid = sid * _NC + cid
        pltpu.sync_copy(zr, dsp.at[pl.ds(sid * _RPT, _RPT)])
        plsc.subcore_barrier()
        lanes = lax.iota(_i32, _L)

        pltpu.sync_copy(src3.at[wid], sidx)
        pltpu.sync_copy(dst3.at[wid], didx)

        def zrow(b, c2):
            ebuf[b, :] = jnp.zeros((_L,), _f32)
            return c2

        lax.fori_loop(0, B, zrow, 0)
        hvecs = [jnp.full((_L,), h, _i32) for h in range(_H)]
        bufs = [(sr0, dr0, ss0, ds0), (sr1, dr1, ss1, ds1)]

        def fire(j, k):
            sr, dr, ss, ds = bufs[k]
            pltpu.async_copy(xw.at[sidx.at[j]], sr, ss)
            pltpu.async_copy(xw.at[didx.at[j]], dr, ds)

        def wait_compute(j, k):
            sr, dr, ss, ds = bufs[k]
            pltpu.make_async_copy(xw.at[sidx.at[j]], sr, ss).wait()
            pltpu.make_async_copy(xw.at[didx.at[j]], dr, ds).wait()

            def group(g, c2):
                bvec = g * _L + lanes

                def chan(c, carry):
                    rot = carry[0]
                    accs = carry[1:]
                    out = []
                    for h in range(_H):
                        cvec = rot + (h * hs)
                        sv = plsc.load_gather(sr, [bvec, cvec])
                        dv = plsc.load_gather(dr, [bvec, cvec])
                        out.append(accs[h] + sv * dv)
                    rot2 = rot + 1
                    rot2 = jnp.where(rot2 >= hs, rot2 - hs, rot2)
                    return (rot2, *out)

                res = lax.fori_loop(
                    0, hs, chan,
                    (lanes,) + tuple(jnp.zeros((_L,), _f32)
                                     for _ in range(_H)))
                for h in range(_H):
                    pv = jnp.exp(res[1 + h] * scale)
                    plsc.store_scatter(ebuf, [bvec, hvecs[h]], pv)
                return c2

            lax.fori_loop(0, B // _L, group, 0)
            ebase = wid * _EW + j * B
            pltpu.sync_copy(ebuf, p_out.at[pl.ds(ebase, B)])
            pltpu.sync_copy(ebuf, dsp.at[didx.at[j]], add=True)

        fire(0, 0)

        def pair(jj, carry):
            j0 = 2 * jj
            fire(j0 + 1, 1)
            wait_compute(j0, 0)
            fire(j0 + 2, 0)
            wait_compute(j0 + 1, 1)
            return carry

        lax.fori_loop(0, (CH - 1) // 2, pair, 0)
        if CH % 2 == 1:
            wait_compute(CH - 1, 0)
        else:
            fire(CH - 1, 1)
            wait_compute(CH - 2, 0)
            wait_compute(CH - 1, 1)
        plsc.subcore_barrier()
        pltpu.sync_copy(dsp.at[pl.ds(sid * _RPT, _RPT)],
                        dpart.at[cid, pl.ds(sid * _RPT, _RPT)])

    return pass1


@functools.partial(
    pl.kernel,
    out_type=jax.ShapeDtypeStruct((_NC, _NPAD, _D1), _f32),
    mesh=_mesh(),
    compiler_params=pltpu.CompilerParams(needs_layout_passes=False, use_tc_tiling_on_sc=False),
    scratch_types=[
        pltpu.VMEM((_B,), _i32),
        pltpu.VMEM((_B,), _i32),
        pltpu.VMEM((_B, _D1), _f32),
        pltpu.VMEM((_B, 16), _f32),
        pltpu.VMEM((_B, _D1), _f32),
        pltpu.VMEM_SHARED((_NPAD, _D1), _f32),
        pltpu.SemaphoreType.DMA,
    ],
)
def _pass2_l1(xw, src, dst, p_in, zr, m_out,
              sidx, didx, srows, pbuf, mbuf, msp, sem):
    cid = lax.axis_index("c")
    sid = lax.axis_index("s")
    wid = sid * _NC + cid
    hvecs = [jnp.full((_L,), h, _i32) for h in range(_H)]
    pltpu.sync_copy(zr, msp.at[pl.ds(sid * _RPT, _RPT)])
    plsc.subcore_barrier()

    def chunk(j, carry):
        ebase = wid * _EW + j * _B
        pltpu.sync_copy(src.at[pl.ds(ebase, _B)], sidx)
        pltpu.sync_copy(dst.at[pl.ds(ebase, _B)], didx)
        pltpu.async_copy(xw.at[sidx], srows, sem).wait()
        pltpu.sync_copy(p_in.at[pl.ds(ebase, _B)], pbuf)

        def edge(b, c2):
            prow = pbuf[b, :]
            for h in range(_H):
                asp = _take16(prow, hvecs[h])
                m = srows[b, pl.ds(h * _L, _L)] * asp
                mbuf[b, pl.ds(h * _L, _L)] = m
            return c2

        lax.fori_loop(0, _B, edge, 0)
        pltpu.sync_copy(mbuf, msp.at[didx], add=True)
        return carry

    lax.fori_loop(0, _CH, chunk, 0)
    plsc.subcore_barrier()
    pltpu.sync_copy(msp.at[pl.ds(sid * _RPT, _RPT)],
                    m_out.at[cid, pl.ds(sid * _RPT, _RPT)])


@functools.partial(
    pl.kernel,
    out_type=(
        jax.ShapeDtypeStruct((_EPAD, 16), _f32),
        jax.ShapeDtypeStruct((_NC, _NPAD, _C2P), _f32),
    ),
    mesh=_mesh(),
    compiler_params=pltpu.CompilerParams(needs_layout_passes=False, use_tc_tiling_on_sc=False),
    scratch_types=[
        pltpu.VMEM((_B,), _i32),
        pltpu.VMEM((_B,), _i32),
        pltpu.VMEM((_B, _D2), _f32),
        pltpu.VMEM((_B, 16), _f32),
        pltpu.VMEM((_B, 128), _f32),
        pltpu.VMEM((_B, 16), _f32),
        pltpu.VMEM((_B, _C2P), _f32),
        pltpu.VMEM_SHARED((_NPAD, _C2P), _f32),
        pltpu.SemaphoreType.DMA,
        pltpu.SemaphoreType.DMA,
    ],
)
def _pass2_l2(xw, src, dst, p_in, invd, zr, a_out, m_out,
              sidx, didx, srows, pbuf, ivd, abuf, mbuf, msp, sem, sem2):
    cid = lax.axis_index("c")
    sid = lax.axis_index("s")
    wid = sid * _NC + cid
    hvecs = [jnp.full((_L,), h, _i32) for h in range(_H)]
    pltpu.sync_copy(zr, msp.at[pl.ds(sid * _RPT, _RPT)])
    plsc.subcore_barrier()

    def chunk(j, carry):
        ebase = wid * _EW + j * _B
        pltpu.sync_copy(src.at[pl.ds(ebase, _B)], sidx)
        pltpu.sync_copy(dst.at[pl.ds(ebase, _B)], didx)
        cp1 = pltpu.async_copy(xw.at[sidx], srows, sem)
        cp2 = pltpu.async_copy(invd.at[didx], ivd, sem2)
        pltpu.sync_copy(p_in.at[pl.ds(ebase, _B)], pbuf)
        cp1.wait()
        cp2.wait()

        def edge(b, c2):
            arow = pbuf[b, :] * ivd[b, pl.ds(0, _L)]
            abuf[b, :] = arow
            macc = [jnp.zeros((_L,), _f32) for _ in range(3)]
            for h in range(_H):
                asp = _take16(arow, hvecs[h])
                for t in range(3):
                    o = h * _C2P + t * _L
                    macc[t] = macc[t] + srows[b, pl.ds(o, _L)] * asp
            for t in range(3):
                mbuf[b, pl.ds(t * _L, _L)] = macc[t]
            return c2

        lax.fori_loop(0, _B, edge, 0)
        pltpu.sync_copy(abuf, a_out.at[pl.ds(ebase, _B)])
        pltpu.sync_copy(mbuf, msp.at[didx], add=True)
        return carry

    lax.fori_loop(0, _CH, chunk, 0)
    plsc.subcore_barrier()
    pltpu.sync_copy(msp.at[pl.ds(sid * _RPT, _RPT)],
                    m_out.at[cid, pl.ds(sid * _RPT, _RPT)])


_RWI = _NPAD // _NW


@functools.partial(
    pl.kernel,
    out_type=jax.ShapeDtypeStruct((_NPAD, 128), _f32),
    mesh=_mesh(),
    compiler_params=pltpu.CompilerParams(needs_layout_passes=False, use_tc_tiling_on_sc=False),
    scratch_types=[
        pltpu.VMEM((_RWI, 16), _f32),
        pltpu.VMEM((_RWI, 16), _f32),
        pltpu.VMEM((_RWI, 128), _f32),
    ],
)
def _invd_table(dp, out, b0, b1, wide):
    cid = lax.axis_index("c")
    sid = lax.axis_index("s")
    wid = sid * _NC + cid
    base = wid * _RWI
    pltpu.sync_copy(dp.at[0, pl.ds(base, _RWI)], b0)
    pltpu.sync_copy(dp.at[1, pl.ds(base, _RWI)], b1)

    def row(r, carry):
        v = 1.0 / (b0[r, :] + b1[r, :] + 1e-16)
        for t in range(8):
            wide[r, pl.ds(t * _L, _L)] = v
        return carry

    lax.fori_loop(0, _RWI, row, 0)
    pltpu.sync_copy(wide, out.at[pl.ds(base, _RWI)])


_RB = 2504


def _mm1_body(x_ref, w_ref, o_ref):
    o_ref[...] = jnp.dot(x_ref[...], w_ref[...],
                         preferred_element_type=_f32)


def _tc_mm1(x_pad, W1):
    return pl.pallas_call(
        _mm1_body,
        grid=(_NPAD // _RB,),
        in_specs=[
            pl.BlockSpec((_RB, _D1), lambda i: (i, 0)),
            pl.BlockSpec((_D1, _D1), lambda i: (0, 0)),
        ],
        out_specs=pl.BlockSpec((_RB, _D1), lambda i: (i, 0)),
        out_shape=jax.ShapeDtypeStruct((_NPAD, _D1), _f32),
    )(x_pad, W1)


def _mm2_body(m0_ref, m1_ref, d0_ref, d1_ref, r_ref, b1_ref, w2_ref, o_ref):
    m0 = m0_ref[0]
    m1 = m1_ref[0]
    den = d0_ref[0] + d1_ref[0] + 1e-16
    den128 = jnp.dot(den, r_ref[...], preferred_element_type=_f32)
    s = (m0 + m1) / den128 + b1_ref[...]
    x1 = jnp.where(s > 0.0, s, jnp.exp(s) - 1.0)
    o_ref[...] = jnp.dot(x1, w2_ref[...], preferred_element_type=_f32)


def _tc_mm2(m1part, dpart1, R, b1, W2p):
    return pl.pallas_call(
        _mm2_body,
        grid=(_NPAD // _RB,),
        in_specs=[
            pl.BlockSpec((1, _RB, _D1), lambda i: (0, i, 0)),
            pl.BlockSpec((1, _RB, _D1), lambda i: (1, i, 0)),
            pl.BlockSpec((1, _RB, 16), lambda i: (0, i, 0)),
            pl.BlockSpec((1, _RB, 16), lambda i: (1, i, 0)),
            pl.BlockSpec((16, _D1), lambda i: (0, 0)),
            pl.BlockSpec((1, _D1), lambda i: (0, 0)),
            pl.BlockSpec((_D1, _D2), lambda i: (0, 0)),
        ],
        out_specs=pl.BlockSpec((_RB, _D2), lambda i: (i, 0)),
        out_shape=jax.ShapeDtypeStruct((_NPAD, _D2), _f32),
    )(m1part, m1part, dpart1, dpart1, R, b1, W2p)


def _copy_body(x_ref, o_ref):
    o_ref[...] = x_ref[...]


def _tc_slice_rows(x, rows, rb):
    w = x.shape[1]
    return pl.pallas_call(
        _copy_body,
        grid=(rows // rb,),
        in_specs=[pl.BlockSpec((rb, w), lambda i: (i, 0))],
        out_specs=pl.BlockSpec((rb, w), lambda i: (i, 0)),
        out_shape=jax.ShapeDtypeStruct((rows, w), _f32),
    )(x)


def _sel_body(x_ref, s_ref, o_ref):
    o_ref[...] = jnp.dot(x_ref[...], s_ref[...], preferred_element_type=_f32)


def _tc_select_cols(x, S, rows, rb):
    w = x.shape[1]
    w2 = S.shape[1]
    return pl.pallas_call(
        _sel_body,
        grid=(rows // rb,),
        in_specs=[
            pl.BlockSpec((rb, w), lambda i: (i, 0)),
            pl.BlockSpec((w, w2), lambda i: (0, 0)),
        ],
        out_specs=pl.BlockSpec((rb, w2), lambda i: (i, 0)),
        out_shape=jax.ShapeDtypeStruct((rows, w2), _f32),
    )(x, S)


def _out2_body(m0_ref, m1_ref, b2_ref, s_ref, o_ref):
    r = (m0_ref[0] + m1_ref[0]) * (1.0 / _H) + b2_ref[...]
    o_ref[...] = jnp.dot(r, s_ref[...], preferred_element_type=_f32)


def _tc_out2(m2part, b2p, S48):
    rb = 400
    return pl.pallas_call(
        _out2_body,
        grid=(_N // rb,),
        in_specs=[
            pl.BlockSpec((1, rb, _C2P), lambda i: (0, i, 0)),
            pl.BlockSpec((1, rb, _C2P), lambda i: (1, i, 0)),
            pl.BlockSpec((1, _C2P), lambda i: (0, 0)),
            pl.BlockSpec((_C2P, _C2), lambda i: (0, 0)),
        ],
        out_specs=pl.BlockSpec((rb, _C2), lambda i: (i, 0)),
        out_shape=jax.ShapeDtypeStruct((_N, _C2), _f32),
    )(m2part, m2part, b2p, S48)


_B1, _CH1 = 128, 81
_B2, _CH2 = 48, 216
_pass1_l1 = _make_pass1(_D1, 1, _SCALE1, _B1, _CH1)
_pass1_l2 = _make_pass1(_D2, 3, _SCALE2, _B2, _CH2)


def kernel(x, edge_index, W1, b1, W2, b2):
    x = x.astype(_f32)
    ei = edge_index.astype(_i32)
    loop = jnp.arange(_N, dtype=_i32)
    npad_e = _EPAD - _E_REAL
    src = jnp.concatenate([ei[0], loop, jnp.zeros((npad_e,), _i32)])
    dst = jnp.concatenate([ei[1], loop, jnp.full((npad_e,), _N, _i32)])

    x_pad = jnp.pad(x, ((0, _NPAD - _N), (0, 0)))
    W2p = jnp.pad(W2.reshape(_D1, _H, _C2).astype(_f32),
                  ((0, 0), (0, 0), (0, _C2P - _C2))).reshape(_D1, _D2)
    b1r = b1.astype(_f32).reshape(1, _D1)
    b2pr = jnp.pad(b2.astype(_f32), (0, _C2P - _C2)).reshape(1, _C2P)

    zr16 = jnp.zeros((_RPT, 16), _f32)
    zr128 = jnp.zeros((_RPT, _D1), _f32)
    zr48 = jnp.zeros((_RPT, _C2P), _f32)

    R = (jnp.repeat(jnp.eye(8, dtype=_f32), 16, axis=1))
    R = jnp.concatenate([R, jnp.zeros((8, _D1), _f32)], axis=0)
    eye48 = jnp.eye(_C2P, dtype=_f32)
    S48 = eye48[:, :_C2]
    Sh2 = jnp.zeros((_D2, _H * _C2), _f32)
    for h in range(_H):
        Sh2 = Sh2.at[h * _C2P:h * _C2P + _C2,
                     h * _C2:(h + 1) * _C2].set(jnp.eye(_C2, dtype=_f32))
    S16 = jnp.eye(16, dtype=_f32)[:, :_H]

    xw1 = _tc_mm1(x_pad, W1.astype(_f32))
    src3a = src.reshape(_NW, _CH1, _B1)
    dst3a = dst.reshape(_NW, _CH1, _B1)
    src3b = src.reshape(_NW, _CH2, _B2)
    dst3b = dst.reshape(_NW, _CH2, _B2)
    p1, dpart1 = _pass1_l1(xw1, src3a, dst3a, zr16)
    m1part = _pass2_l1(xw1, src, dst, p1, zr128)
    if isinstance(m1part, (tuple, list)):
        m1part = m1part[0]
    xw2 = _tc_mm2(m1part, dpart1, R, b1r, W2p)
    p2, dpart2 = _pass1_l2(xw2, src3b, dst3b, zr16)
    invd2 = _invd_table(dpart2)
    a2_16, m2part = _pass2_l2(xw2, src, dst, p2, invd2, zr48)

    out2 = _tc_out2(m2part, b2pr, S48)
    h1 = _tc_slice_rows(xw1, _N, 400).reshape(_N, _H, _C1)
    h2 = _tc_select_cols(xw2, Sh2, _N, 400).reshape(_N, _H, _C2)
    a2 = _tc_select_cols(a2_16, S16, _E_REAL, 2000)
    return (out2, h1, h2, a2)
